# baseline (speedup 1.0000x reference)
import jax
import jax.numpy as jnp
from jax import lax
from jax.experimental import pallas as pl

_POOLED_H = 7
_POOLED_W = 7
_POOLED_L = 4
_TEMPORAL_SCALE = 0.125


def _roi_bins_kernel(rois_ref, out_ref, *, num_l):
    rois = rois_ref[...]

    start_l = jnp.round(rois[:, 5:6] * _TEMPORAL_SCALE).astype(jnp.int32)
    end_l = jnp.round(rois[:, 6:7] * _TEMPORAL_SCALE).astype(jnp.int32)
    roi_length = jnp.maximum(end_l - start_l + 1, 1)
    bin_size_l = roi_length.astype(jnp.float32) * (1.0 / _POOLED_L)

    pl_idx = lax.broadcasted_iota(jnp.int32, (1, _POOLED_L), 1).astype(
        jnp.float32
    )
    lstart = jnp.clip(
        jnp.floor(pl_idx * bin_size_l).astype(jnp.int32) + start_l, 0, num_l
    )
    lend = jnp.clip(
        jnp.floor((pl_idx + 1.0) * bin_size_l).astype(jnp.int32) + start_l, 0, num_l
    )
    is_empty = lstart <= lend

    val = jnp.where(is_empty, 0.0, -jnp.inf)

    out_ref[...] = jnp.broadcast_to(
        val[:, :, None, None], out_ref.shape
    )


def kernel(features, rois):
    B, C, L, H, W = features.shape
    num_rois = rois.shape[0]

    pooled = pl.pallas_call(
        lambda r, o: _roi_bins_kernel(r, o, num_l=L),
        in_specs=[pl.BlockSpec(rois.shape, lambda: (0, 0))],
        out_specs=pl.BlockSpec(
            (num_rois, _POOLED_L, _POOLED_H, _POOLED_W), lambda: (0, 0, 0, 0)
        ),
        out_shape=jax.ShapeDtypeStruct(
            (num_rois, _POOLED_L, _POOLED_H, _POOLED_W), jnp.float32
        ),
    )(rois)

    return jnp.broadcast_to(
        pooled[:, None, :, :, :],
        (num_rois, C, _POOLED_L, _POOLED_H, _POOLED_W),
    )


# speedup vs baseline: 1.2674x; 1.2674x over previous
import jax
import jax.numpy as jnp
from jax import lax
from jax.experimental import pallas as pl

_POOLED_H = 7
_POOLED_W = 7
_POOLED_L = 4
_TEMPORAL_SCALE = 0.125


def _roi_bins_kernel(rois_ref, out_ref, *, num_l):
    rois = rois_ref[...]

    start_l = jnp.round(rois[:, 5:6] * _TEMPORAL_SCALE).astype(jnp.int32)
    end_l = jnp.round(rois[:, 6:7] * _TEMPORAL_SCALE).astype(jnp.int32)
    roi_length = jnp.maximum(end_l - start_l + 1, 1)
    bin_size_l = roi_length.astype(jnp.float32) * (1.0 / _POOLED_L)

    pl_idx = lax.broadcasted_iota(jnp.int32, (1, _POOLED_L), 1).astype(
        jnp.float32
    )
    lstart = jnp.clip(
        jnp.floor(pl_idx * bin_size_l).astype(jnp.int32) + start_l, 0, num_l
    )
    lend = jnp.clip(
        jnp.floor((pl_idx + 1.0) * bin_size_l).astype(jnp.int32) + start_l, 0, num_l
    )
    is_empty = lstart <= lend

    val = jnp.where(is_empty, 0.0, -jnp.inf)

    hw = _POOLED_H * _POOLED_W
    j = lax.broadcasted_iota(jnp.int32, (1, _POOLED_L * hw), 1) // hw
    out_ref[...] = jnp.where(
        j == 0,
        val[:, 0:1],
        jnp.where(j == 1, val[:, 1:2], jnp.where(j == 2, val[:, 2:3], val[:, 3:4])),
    )


def kernel(features, rois):
    B, C, L, H, W = features.shape
    num_rois = rois.shape[0]

    flat = _POOLED_L * _POOLED_H * _POOLED_W
    pooled = pl.pallas_call(
        lambda r, o: _roi_bins_kernel(r, o, num_l=L),
        in_specs=[pl.BlockSpec(rois.shape, lambda: (0, 0))],
        out_specs=pl.BlockSpec((num_rois, flat), lambda: (0, 0)),
        out_shape=jax.ShapeDtypeStruct((num_rois, flat), jnp.float32),
    )(rois)

    pooled = pooled.reshape(num_rois, _POOLED_L, _POOLED_H, _POOLED_W)
    return jnp.broadcast_to(
        pooled[:, None, :, :, :],
        (num_rois, C, _POOLED_L, _POOLED_H, _POOLED_W),
    )
